# baseline (speedup 1.0000x reference)
import functools

import jax
import jax.numpy as jnp
from jax import lax
from jax.experimental import pallas as pl
from jax.experimental.pallas import tpu as pltpu
from jax.experimental.pallas import tpu_sc as plsc

VOCAB = 1000000
HIDDEN = 64
NUM_CLASSES = 3
BATCH = 4096
SEQ = 200

NC = 2
NS = 16
NW = NC * NS
ROWS_PER_W = BATCH // NW
HALF = SEQ // 2
IDXROWS_PER_W = 2 * ROWS_PER_W


NBUF = 4


def _sc_kernel(ids_hbm, table_hbm, wt_hbm, b_hbm, out_hbm,
               idx_v, rows_v, w_v, b_v, out_v, *sems):
  wid = lax.axis_index("s") * NC + lax.axis_index("c")
  ibase = wid * IDXROWS_PER_W
  obase = wid * ROWS_PER_W

  pltpu.sync_copy(ids_hbm.at[pl.ds(ibase, IDXROWS_PER_W)], idx_v)
  pltpu.sync_copy(wt_hbm, w_v)
  pltpu.sync_copy(b_hbm, b_v)

  inv = jnp.float32(1.0 / SEQ)
  bvec = b_v[pl.ds(0, 16)]
  iot = lax.iota(jnp.int32, 16)

  def fire(slot, i):
    pltpu.async_copy(table_hbm.at[idx_v.at[2 * i]],
                     rows_v.at[pl.ds(slot * SEQ, HALF)], sems[slot])
    pltpu.async_copy(table_hbm.at[idx_v.at[2 * i + 1]],
                     rows_v.at[pl.ds(slot * SEQ + HALF, HALF)], sems[slot])

  def wait_slot(slot):
    pltpu.make_async_copy(table_hbm.at[idx_v.at[0]],
                          rows_v.at[pl.ds(slot * SEQ, HALF)],
                          sems[slot]).wait()
    pltpu.make_async_copy(table_hbm.at[idx_v.at[0]],
                          rows_v.at[pl.ds(slot * SEQ + HALF, HALF)],
                          sems[slot]).wait()

  for b in range(NBUF - 1):
    fire(b, b)

  def outer(j, _):
    for b in range(NBUF):
      i = j * NBUF + b
      nxt = i + NBUF - 1

      @pl.when(nxt < ROWS_PER_W)
      def _():
        fire((b + NBUF - 1) % NBUF, nxt)

      wait_slot(b)
      base = b * SEQ
      zero = jnp.zeros((16,), jnp.float32)

      def acc_body(k, acc):
        a0, a1, a2, a3 = acc
        a0 = a0 + rows_v[base + k, pl.ds(0, 16)]
        a1 = a1 + rows_v[base + k, pl.ds(16, 16)]
        a2 = a2 + rows_v[base + k, pl.ds(32, 16)]
        a3 = a3 + rows_v[base + k, pl.ds(48, 16)]
        return (a0, a1, a2, a3)

      a0, a1, a2, a3 = lax.fori_loop(0, SEQ, acc_body,
                                     (zero, zero, zero, zero), unroll=8)

      logits = []
      for c in range(NUM_CLASSES):
        s = (jnp.sum(a0 * w_v[c, pl.ds(0, 16)]) +
             jnp.sum(a1 * w_v[c, pl.ds(16, 16)]) +
             jnp.sum(a2 * w_v[c, pl.ds(32, 16)]) +
             jnp.sum(a3 * w_v[c, pl.ds(48, 16)]))
        logits.append(s * inv + bvec[c])
      lv = jnp.where(iot == 0, logits[0],
                     jnp.where(iot == 1, logits[1], logits[2]))
      plsc.store_scatter(out_v, [NUM_CLASSES * i + iot], lv,
                         mask=iot < NUM_CLASSES)
    return 0

  lax.fori_loop(0, ROWS_PER_W // NBUF, outer, 0)

  pltpu.sync_copy(out_v, out_hbm.at[pl.ds(obase * NUM_CLASSES,
                                          ROWS_PER_W * NUM_CLASSES)])


@jax.jit
def _run(ids2, table, wt, bpad):
  mesh = plsc.VectorSubcoreMesh(core_axis_name="c", subcore_axis_name="s")
  f = functools.partial(
      pl.kernel,
      out_type=jax.ShapeDtypeStruct((BATCH * NUM_CLASSES,), jnp.float32),
      mesh=mesh,
      scratch_types=[
          pltpu.VMEM((IDXROWS_PER_W, HALF), jnp.int32),
          pltpu.VMEM((NBUF * SEQ, HIDDEN), jnp.float32),
          pltpu.VMEM((NUM_CLASSES, HIDDEN), jnp.float32),
          pltpu.VMEM((16,), jnp.float32),
          pltpu.VMEM((ROWS_PER_W * NUM_CLASSES,), jnp.float32),
      ] + [pltpu.SemaphoreType.DMA] * NBUF,
      compiler_params=pltpu.CompilerParams(needs_layout_passes=False,
                                           use_tc_tiling_on_sc=False),
  )(_sc_kernel)
  return f(ids2, table, wt, bpad)


def kernel(input_ids, emb_table, W, b):
  ids2 = input_ids.astype(jnp.int32).reshape(BATCH * 2, HALF)
  wt = W.T
  bpad = jnp.pad(b.astype(jnp.float32), (0, 16 - NUM_CLASSES))
  return _run(ids2, emb_table, wt, bpad).reshape(BATCH, NUM_CLASSES)
